# Initial kernel scaffold; baseline (speedup 1.0000x reference)
import jax
import jax.numpy as jnp
from jax.experimental import pallas as pl


def kernel(features, y, codebooks, Win1, bin1, Win2, bin2, g_in, b_in, Wout1, bout1, Wout2, bout2, g_out, b_out):
    raise NotImplementedError("write your pallas kernel here")


# speedup vs baseline: 1.5305x; 1.5305x over previous
import functools

import jax
import jax.numpy as jnp
from jax import lax
from jax.experimental import pallas as pl
from jax.experimental.pallas import tpu as pltpu
from jax.experimental.pallas import tpu_sc as plsc

NUM_EMB = 8192
CODE_DIM = 256
EMB_DIM = 768
KB = 1024
NKB = NUM_EMB // KB

_NC = 2
_NS = 16
_NW = _NC * _NS


def _ct(a, b):
    return lax.dot_general(a, b, (((1,), (1,)), ((), ())),
                           preferred_element_type=jnp.float32)


def _layer_norm(x, g, b, eps=1e-5):
    mu = jnp.mean(x, axis=1, keepdims=True)
    var = jnp.mean((x - mu) ** 2, axis=1, keepdims=True)
    return (x - mu) / jnp.sqrt(var + eps) * g + b


def _proj_in_body(x_ref, w1_ref, b1_ref, w2_ref, b2_ref, g_ref, bb_ref, o_ref):
    x = x_ref[0]
    h1 = jnp.maximum(_ct(x, w1_ref[...]) + b1_ref[...], 0.0)
    h2 = _ct(h1, w2_ref[...]) + b2_ref[...]
    o_ref[0] = _layer_norm(h2, g_ref[...], bb_ref[...])


def _proj_out_body(q_ref, w1_ref, b1_ref, w2_ref, b2_ref, g_ref, bb_ref, o_ref):
    q = q_ref[0]
    r1 = jnp.maximum(_ct(q, w1_ref[...]) + b1_ref[...], 0.0)
    r2 = _ct(r1, w2_ref[...]) + b2_ref[...]
    o_ref[0] = _layer_norm(r2, g_ref[...], bb_ref[...])


def _argmin_body(y_ref, h_ref, cb_ref, ei_ref, gi_ref, rmin, rarg, *, T):
    b = pl.program_id(0)
    kb = pl.program_id(1)
    h = h_ref[0]
    cb = cb_ref[0]
    hh = jnp.sum(h * h, axis=1, keepdims=True)
    cc = jnp.sum(cb * cb, axis=1, keepdims=True).T
    s = _ct(h, cb)
    d2 = (hh + cc) - 2.0 * s
    iota = lax.broadcasted_iota(jnp.int32, (T, KB), 1) + kb * KB
    bmin = jnp.min(d2, axis=1, keepdims=True)
    barg = jnp.min(jnp.where(d2 == bmin, iota, jnp.int32(2**31 - 1)),
                   axis=1, keepdims=True)

    @pl.when(kb == 0)
    def _():
        rmin[...] = bmin
        rarg[...] = barg

    @pl.when(kb > 0)
    def _():
        better = bmin < rmin[...]
        rarg[...] = jnp.where(better, barg, rarg[...])
        rmin[...] = jnp.where(better, bmin, rmin[...])

    ei_ref[0] = rarg[...]
    gi_ref[0] = rarg[...] + y_ref[b] * NUM_EMB


def _enc_body(ei_ref, o_ref, *, T):
    kb = pl.program_id(1)
    iota = lax.broadcasted_iota(jnp.int32, (T, KB), 1) + kb * KB
    o_ref[0] = jnp.where(ei_ref[0] == iota, jnp.float32(1.0), jnp.float32(0.0))


def _make_sc_gather(n_rows, d):
    bpw = n_rows // _NW
    mesh = plsc.VectorSubcoreMesh(core_axis_name="c", subcore_axis_name="s")

    @functools.partial(
        pl.kernel, mesh=mesh,
        out_type=jax.ShapeDtypeStruct((n_rows, d), jnp.float32),
        scratch_types=[
            pltpu.VMEM((bpw,), jnp.int32),
            pltpu.VMEM((bpw, d), jnp.float32),
            pltpu.SemaphoreType.DMA,
        ],
    )
    def gather_k(table_hbm, idx_hbm, out_hbm, idx_v, rows_v, sem):
        wid = lax.axis_index("s") * _NC + lax.axis_index("c")
        base = wid * bpw
        pltpu.sync_copy(idx_hbm.at[pl.ds(base, bpw)], idx_v)
        pltpu.async_copy(table_hbm.at[idx_v], rows_v, sem).wait()
        pltpu.sync_copy(rows_v, out_hbm.at[pl.ds(base, bpw)])

    return gather_k


def kernel(features, y, codebooks, Win1, bin1, Win2, bin2, g_in, b_in,
           Wout1, bout1, Wout2, bout2, g_out, b_out):
    B, C, H, W = features.shape
    T = H * W
    f = features.reshape(B, C, T).transpose(0, 2, 1)

    bin1_2 = bin1.reshape(1, -1)
    bin2_2 = bin2.reshape(1, -1)
    g_in_2 = g_in.reshape(1, -1)
    b_in_2 = b_in.reshape(1, -1)
    bout1_2 = bout1.reshape(1, -1)
    bout2_2 = bout2.reshape(1, -1)
    g_out_2 = g_out.reshape(1, -1)
    b_out_2 = b_out.reshape(1, -1)

    h = pl.pallas_call(
        _proj_in_body,
        grid=(B,),
        in_specs=[
            pl.BlockSpec((1, T, C), lambda b: (b, 0, 0)),
            pl.BlockSpec((C, C), lambda b: (0, 0)),
            pl.BlockSpec((1, C), lambda b: (0, 0)),
            pl.BlockSpec((CODE_DIM, C), lambda b: (0, 0)),
            pl.BlockSpec((1, CODE_DIM), lambda b: (0, 0)),
            pl.BlockSpec((1, CODE_DIM), lambda b: (0, 0)),
            pl.BlockSpec((1, CODE_DIM), lambda b: (0, 0)),
        ],
        out_specs=pl.BlockSpec((1, T, CODE_DIM), lambda b: (b, 0, 0)),
        out_shape=jax.ShapeDtypeStruct((B, T, CODE_DIM), jnp.float32),
    )(f, Win1, bin1_2, Win2, bin2_2, g_in_2, b_in_2)

    ei3, gi3 = pl.pallas_call(
        functools.partial(_argmin_body, T=T),
        grid_spec=pltpu.PrefetchScalarGridSpec(
            num_scalar_prefetch=1,
            grid=(B, NKB),
            in_specs=[
                pl.BlockSpec((1, T, CODE_DIM), lambda b, kb, y_r: (b, 0, 0)),
                pl.BlockSpec((1, KB, CODE_DIM), lambda b, kb, y_r: (y_r[b], kb, 0)),
            ],
            out_specs=[
                pl.BlockSpec((1, T, 1), lambda b, kb, y_r: (b, 0, 0)),
                pl.BlockSpec((1, T, 1), lambda b, kb, y_r: (b, 0, 0)),
            ],
            scratch_shapes=[
                pltpu.VMEM((T, 1), jnp.float32),
                pltpu.VMEM((T, 1), jnp.int32),
            ],
        ),
        out_shape=[
            jax.ShapeDtypeStruct((B, T, 1), jnp.int32),
            jax.ShapeDtypeStruct((B, T, 1), jnp.int32),
        ],
    )(y.astype(jnp.int32), h, codebooks)

    enc = pl.pallas_call(
        functools.partial(_enc_body, T=T),
        grid=(B, NKB),
        in_specs=[pl.BlockSpec((1, T, 1), lambda b, kb: (b, 0, 0))],
        out_specs=pl.BlockSpec((1, T, KB), lambda b, kb: (b, 0, kb)),
        out_shape=jax.ShapeDtypeStruct((B, T, NUM_EMB), jnp.float32),
    )(ei3)

    gidx = gi3.reshape(-1)
    table = codebooks.reshape(-1, CODE_DIM)
    q = _make_sc_gather(B * T, CODE_DIM)(table, gidx)

    quant = pl.pallas_call(
        _proj_out_body,
        grid=(B,),
        in_specs=[
            pl.BlockSpec((1, T, CODE_DIM), lambda b: (b, 0, 0)),
            pl.BlockSpec((C, CODE_DIM), lambda b: (0, 0)),
            pl.BlockSpec((1, C), lambda b: (0, 0)),
            pl.BlockSpec((C, C), lambda b: (0, 0)),
            pl.BlockSpec((1, C), lambda b: (0, 0)),
            pl.BlockSpec((1, C), lambda b: (0, 0)),
            pl.BlockSpec((1, C), lambda b: (0, 0)),
        ],
        out_specs=pl.BlockSpec((1, T, C), lambda b: (b, 0, 0)),
        out_shape=jax.ShapeDtypeStruct((B, T, C), jnp.float32),
    )(q.reshape(B, T, CODE_DIM), Wout1, bout1_2, Wout2, bout2_2, g_out_2, b_out_2)

    return (quant, ei3.reshape(-1, 1), enc)
